# baseline (speedup 1.0000x reference)
import jax
import jax.numpy as jnp
from jax.experimental import pallas as pl
from jax.experimental.pallas import tpu as pltpu

_SIZE = 2048
_SCALE = 1.0 / (_SIZE * _SIZE)
_TA = 512
_S = _SIZE // _TA


def _stencil_kernel(x2_ref, x3_ref, x4_ref, out_ref, wprev_ref, lcol_ref):
    s = pl.program_id(0)
    b = jnp.minimum(s, _S - 1)

    r1 = jax.lax.broadcasted_iota(jnp.int32, (_SIZE, 1), 0)
    rmaskf1 = jnp.where((r1 >= 1) & (r1 <= _SIZE - 2), 1.0, 0.0)

    nraw = jnp.maximum(jnp.maximum(x2_ref[0, :, 0:1] * 1000.0,
                                   x3_ref[0, :, 0:1] * 100.0),
                       x4_ref[0, :, 0:1] * 10.0)
    ncol = jnp.where(s == _S, 0.0, nraw * rmaskf1)

    @pl.when(s > 0)
    def _():
        wc = wprev_ref[...]
        zrow = jnp.zeros((1, _TA), jnp.float32)
        t1 = jnp.concatenate([wc[:, 1:], ncol], axis=1)
        t2 = jnp.concatenate([wc[1:], zrow], axis=0)
        t3 = jnp.concatenate([zrow, wc[:-1]], axis=0)
        t4 = jnp.concatenate([lcol_ref[...], wc[:, :-1]], axis=1)
        p = jnp.where(t1 != 0.0, t1,
            jnp.where(t2 != 0.0, t2,
            jnp.where(t3 != 0.0, t3,
            jnp.where(t4 != 0.0, t4, _SCALE))))
        pb = p.astype(jnp.bfloat16)
        eye = jnp.eye(128, dtype=jnp.bfloat16)
        dn = (((0,), (0,)), ((), ()))
        pt = jnp.concatenate(
            [jax.lax.dot_general(pb[128 * k:128 * (k + 1), :], eye, dn,
                                 preferred_element_type=jnp.float32)
             for k in range(_SIZE // 128)], axis=1)
        out_ref[...] = pt.reshape(_TA, 16, 128)

    @pl.when(s < _S)
    def _():
        wcur = jnp.maximum(jnp.maximum(x2_ref[0] * 1000.0,
                                       x3_ref[0] * 100.0),
                           x4_ref[0] * 10.0)
        c1 = b * _TA + jax.lax.broadcasted_iota(jnp.int32, (1, _TA), 1)
        cmaskf1 = jnp.where((c1 >= 1) & (c1 <= _SIZE - 2), 1.0, 0.0)
        wcur = wcur * rmaskf1 * cmaskf1
        lcol_ref[...] = jnp.where(s == 0, 0.0, wprev_ref[:, _TA - 1:_TA])
        wprev_ref[...] = wcur


def kernel(x):
    x = x.reshape(5, _SIZE, _SIZE)
    out = pl.pallas_call(
        _stencil_kernel,
        grid=(_S + 1,),
        in_specs=[
            pl.BlockSpec((1, _SIZE, _TA), lambda s: (2, 0, jnp.minimum(s, _S - 1))),
            pl.BlockSpec((1, _SIZE, _TA), lambda s: (3, 0, jnp.minimum(s, _S - 1))),
            pl.BlockSpec((1, _SIZE, _TA), lambda s: (4, 0, jnp.minimum(s, _S - 1))),
        ],
        out_specs=pl.BlockSpec((_TA, 16, 128), lambda s: (jnp.maximum(s - 1, 0), 0, 0)),
        out_shape=jax.ShapeDtypeStruct((_SIZE, 16, 128), jnp.float32),
        scratch_shapes=[
            pltpu.VMEM((_SIZE, _TA), jnp.float32),
            pltpu.VMEM((_SIZE, 1), jnp.float32),
        ],
    )(x, x, x)

    value = jnp.array([0], dtype=jnp.int32)
    return (value, out.reshape(_SIZE * _SIZE))


# speedup vs baseline: 1.2316x; 1.2316x over previous
import jax
import jax.numpy as jnp
from jax.experimental import pallas as pl
from jax.experimental.pallas import tpu as pltpu

_SIZE = 2048
_SCALE = 1.0 / (_SIZE * _SIZE)
_TA = 512
_S = _SIZE // _TA


def _stencil_kernel(x2_ref, x3_ref, x4_ref, out_ref, wprev_ref, lcol_ref):
    s = pl.program_id(0)
    b = jnp.minimum(s, _S - 1)

    r1 = jax.lax.broadcasted_iota(jnp.int32, (_SIZE, 1), 0)
    rmaskf1 = jnp.where((r1 >= 1) & (r1 <= _SIZE - 2), 1.0, 0.0)

    nraw = jnp.maximum(jnp.maximum(x2_ref[0, :, 0:1] * 1000.0,
                                   x3_ref[0, :, 0:1] * 100.0),
                       x4_ref[0, :, 0:1] * 10.0)
    ncol = jnp.where(s == _S, 0.0, nraw * rmaskf1).astype(jnp.bfloat16)

    @pl.when(s > 0)
    def _():
        wc = wprev_ref[...]
        zrow = jnp.zeros((1, _TA), jnp.bfloat16)
        t1 = jnp.concatenate([wc[:, 1:], ncol], axis=1)
        t2 = jnp.concatenate([wc[1:], zrow], axis=0)
        t3 = jnp.concatenate([zrow, wc[:-1]], axis=0)
        t4 = jnp.concatenate([lcol_ref[...], wc[:, :-1]], axis=1)
        zero = jnp.bfloat16(0.0)
        p = jnp.where(t1 != zero, t1,
            jnp.where(t2 != zero, t2,
            jnp.where(t3 != zero, t3,
            jnp.where(t4 != zero, t4, jnp.bfloat16(_SCALE)))))
        eye = jnp.eye(128, dtype=jnp.bfloat16)
        dn = (((0,), (0,)), ((), ()))
        pt = jnp.concatenate(
            [jax.lax.dot_general(p[128 * k:128 * (k + 1), :], eye, dn,
                                 preferred_element_type=jnp.float32)
             for k in range(_SIZE // 128)], axis=1)
        out_ref[...] = pt.reshape(_TA, 16, 128)

    @pl.when(s < _S)
    def _():
        wcur = jnp.maximum(jnp.maximum(x2_ref[0] * 1000.0,
                                       x3_ref[0] * 100.0),
                           x4_ref[0] * 10.0)
        c1 = b * _TA + jax.lax.broadcasted_iota(jnp.int32, (1, _TA), 1)
        cmaskf1 = jnp.where((c1 >= 1) & (c1 <= _SIZE - 2), 1.0, 0.0)
        wcur = wcur * rmaskf1 * cmaskf1
        lcol_ref[...] = jnp.where(s == 0, jnp.bfloat16(0.0),
                                  wprev_ref[:, _TA - 1:_TA])
        wprev_ref[...] = wcur.astype(jnp.bfloat16)


def kernel(x):
    x = x.reshape(5, _SIZE, _SIZE)
    out = pl.pallas_call(
        _stencil_kernel,
        grid=(_S + 1,),
        in_specs=[
            pl.BlockSpec((1, _SIZE, _TA), lambda s: (2, 0, jnp.minimum(s, _S - 1))),
            pl.BlockSpec((1, _SIZE, _TA), lambda s: (3, 0, jnp.minimum(s, _S - 1))),
            pl.BlockSpec((1, _SIZE, _TA), lambda s: (4, 0, jnp.minimum(s, _S - 1))),
        ],
        out_specs=pl.BlockSpec((_TA, 16, 128), lambda s: (jnp.maximum(s - 1, 0), 0, 0)),
        out_shape=jax.ShapeDtypeStruct((_SIZE, 16, 128), jnp.float32),
        scratch_shapes=[
            pltpu.VMEM((_SIZE, _TA), jnp.bfloat16),
            pltpu.VMEM((_SIZE, 1), jnp.bfloat16),
        ],
    )(x, x, x)

    value = jnp.array([0], dtype=jnp.int32)
    return (value, out.reshape(_SIZE * _SIZE))


# speedup vs baseline: 1.4038x; 1.1399x over previous
import jax
import jax.numpy as jnp
from jax.experimental import pallas as pl
from jax.experimental.pallas import tpu as pltpu

_SIZE = 2048
_SCALE = 1.0 / (_SIZE * _SIZE)
_TA = 512
_S = _SIZE // _TA


def _stencil_kernel(x2_ref, x3_ref, x4_ref, n2_ref, n3_ref, n4_ref,
                    out_ref, lcol_ref):
    s = pl.program_id(0)

    r1 = jax.lax.broadcasted_iota(jnp.int32, (_SIZE, 1), 0)
    rin = (r1 >= 1) & (r1 <= _SIZE - 2)
    rmask10 = jnp.where(rin, 10.0, 0.0)

    wraw = jnp.maximum(jnp.maximum(x2_ref[0] * 100.0, x3_ref[0] * 10.0),
                       x4_ref[0])
    c1 = s * _TA + jax.lax.broadcasted_iota(jnp.int32, (1, _TA), 1)
    cmaskf1 = jnp.where((c1 >= 1) & (c1 <= _SIZE - 2), 1.0, 0.0)
    wc = ((wraw * rmask10) * cmaskf1).astype(jnp.bfloat16)

    nraw = jnp.maximum(jnp.maximum(n2_ref[0, :, 0:1] * 100.0,
                                   n3_ref[0, :, 0:1] * 10.0),
                       n4_ref[0, :, 0:1])
    ncol = jnp.where(s == _S - 1, 0.0, nraw * rmask10).astype(jnp.bfloat16)
    lcol = jnp.where(s == 0, jnp.bfloat16(0.0), lcol_ref[...])

    zrow = jnp.zeros((1, _TA), jnp.bfloat16)
    t1 = jnp.concatenate([wc[:, 1:], ncol], axis=1)
    t2 = jnp.concatenate([wc[1:], zrow], axis=0)
    t3 = jnp.concatenate([zrow, wc[:-1]], axis=0)
    t4 = jnp.concatenate([lcol, wc[:, :-1]], axis=1)
    zero = jnp.bfloat16(0.0)
    p = jnp.where(t1 != zero, t1,
        jnp.where(t2 != zero, t2,
        jnp.where(t3 != zero, t3,
        jnp.where(t4 != zero, t4, jnp.bfloat16(_SCALE)))))
    eye = jnp.eye(128, dtype=jnp.bfloat16)
    dn = (((0,), (0,)), ((), ()))
    pt = jnp.concatenate(
        [jax.lax.dot_general(p[128 * k:128 * (k + 1), :], eye, dn,
                             preferred_element_type=jnp.float32)
         for k in range(_SIZE // 128)], axis=1)
    out_ref[...] = pt.reshape(_TA, 16, 128)

    lcol_ref[...] = wc[:, _TA - 1:_TA]


def kernel(x):
    x = x.reshape(5, _SIZE, _SIZE)
    main = lambda c: pl.BlockSpec((1, _SIZE, _TA), lambda s, c=c: (c, 0, s))
    nxt = lambda c: pl.BlockSpec(
        (1, _SIZE, 128),
        lambda s, c=c: (c, 0, (_TA // 128) * jnp.minimum(s + 1, _S - 1)))
    out = pl.pallas_call(
        _stencil_kernel,
        grid=(_S,),
        in_specs=[main(2), main(3), main(4), nxt(2), nxt(3), nxt(4)],
        out_specs=pl.BlockSpec((_TA, 16, 128), lambda s: (s, 0, 0)),
        out_shape=jax.ShapeDtypeStruct((_SIZE, 16, 128), jnp.float32),
        scratch_shapes=[
            pltpu.VMEM((_SIZE, 1), jnp.bfloat16),
        ],
    )(x, x, x, x, x, x)

    value = jnp.array([0], dtype=jnp.int32)
    return (value, out.reshape(_SIZE * _SIZE))


# speedup vs baseline: 1.4063x; 1.0018x over previous
import jax
import jax.numpy as jnp
from jax.experimental import pallas as pl
from jax.experimental.pallas import tpu as pltpu

_SIZE = 2048
_SCALE = 1.0 / (_SIZE * _SIZE)
_TR = 512
_S = _SIZE // _TR


def _stencil_kernel(x2_ref, x3_ref, x4_ref, n2_ref, n3_ref, n4_ref,
                    out_ref, prow_ref):
    s = pl.program_id(0)

    c1 = jax.lax.broadcasted_iota(jnp.int32, (1, _SIZE), 1)
    cmask10 = jnp.where((c1 >= 1) & (c1 <= _SIZE - 2), 10.0, 0.0)

    wraw = jnp.maximum(jnp.maximum(x2_ref[0] * 100.0, x3_ref[0] * 10.0),
                       x4_ref[0])
    r1 = s * _TR + jax.lax.broadcasted_iota(jnp.int32, (_TR, 1), 0)
    rmaskf = jnp.where((r1 >= 1) & (r1 <= _SIZE - 2), 1.0, 0.0)
    wc = ((wraw * cmask10) * rmaskf).astype(jnp.bfloat16)

    nraw = jnp.maximum(jnp.maximum(n2_ref[0, 0:1, :] * 100.0,
                                   n3_ref[0, 0:1, :] * 10.0),
                       n4_ref[0, 0:1, :])
    hnext = jnp.where(s == _S - 1, 0.0, nraw * cmask10).astype(jnp.bfloat16)
    hprev = jnp.where(s == 0, jnp.bfloat16(0.0), prow_ref[...])

    zcol = jnp.zeros((_TR, 1), jnp.bfloat16)
    t1 = jnp.concatenate([wc[:, 1:], zcol], axis=1)
    t2 = jnp.concatenate([wc[1:], hnext], axis=0)
    t3 = jnp.concatenate([hprev, wc[:-1]], axis=0)
    t4 = jnp.concatenate([zcol, wc[:, :-1]], axis=1)
    zero = jnp.bfloat16(0.0)
    p = jnp.where(t1 != zero, t1,
        jnp.where(t2 != zero, t2,
        jnp.where(t3 != zero, t3,
        jnp.where(t4 != zero, t4, jnp.bfloat16(_SCALE)))))
    eye = jnp.eye(128, dtype=jnp.bfloat16)
    dn = (((0,), (0,)), ((), ()))
    pt = jnp.concatenate(
        [jax.lax.dot_general(p[128 * k:128 * (k + 1), :], eye, dn,
                             preferred_element_type=jnp.float32)
         for k in range(_TR // 128)], axis=1)
    v = pt.reshape(_SIZE, _TR // 128, 128)

    @pl.when(s % 2 == 0)
    def _():
        out_ref[:, 0, 0:4, :] = v

    @pl.when(s % 2 == 1)
    def _():
        out_ref[:, 0, 4:8, :] = v

    prow_ref[...] = wc[_TR - 1:_TR, :]


def kernel(x):
    x = x.reshape(5, _SIZE, _SIZE)
    main = lambda c: pl.BlockSpec((1, _TR, _SIZE), lambda s, c=c: (c, s, 0))
    nxt = lambda c: pl.BlockSpec(
        (1, 8, _SIZE),
        lambda s, c=c: (c, (_TR // 8) * jnp.minimum(s + 1, _S - 1), 0))
    out = pl.pallas_call(
        _stencil_kernel,
        grid=(_S,),
        in_specs=[main(2), main(3), main(4), nxt(2), nxt(3), nxt(4)],
        out_specs=pl.BlockSpec((_SIZE, 1, 8, 128), lambda s: (0, s // 2, 0, 0)),
        out_shape=jax.ShapeDtypeStruct((_SIZE, 2, 8, 128), jnp.float32),
        scratch_shapes=[
            pltpu.VMEM((1, _SIZE), jnp.bfloat16),
        ],
    )(x, x, x, x, x, x)

    value = jnp.array([0], dtype=jnp.int32)
    return (value, out.reshape(_SIZE * _SIZE))
